# Initial kernel scaffold; baseline (speedup 1.0000x reference)
import jax
import jax.numpy as jnp
from jax.experimental import pallas as pl


def kernel(g, enc_W0, enc_b0, enc_W1, enc_b1, enc_eps, rho_W0, rho_b0, rho_W1, rho_b1):
    raise NotImplementedError("write your pallas kernel here")


# speedup vs baseline: 1.9671x; 1.9671x over previous
import functools

import jax
import jax.numpy as jnp
from jax.experimental import pallas as pl
from jax.experimental.pallas import tpu as pltpu

N = 512
M = 4
D = 16
HID = 32
OUT = 16
BU = 32
GRID = N // BU


def _body(eps_ref, g_ref, w0_ref, b0_ref, w1_ref, b1_ref,
          rw0_ref, rb0_ref, rw1_ref, rb1_ref, out_ref,
          a_scr, x_scr):
    step = pl.program_id(0)
    u0 = step * BU

    blk = g_ref[...]
    x_scr[pl.ds(u0, BU), :] = blk[:, 0, :]
    for c in range(M):
        a_scr[c, pl.ds(u0, BU), :] = jnp.sum(
            blk[:, :, 16 * c:16 * (c + 1)], axis=-1) * (1.0 / D)

    @pl.when(step == GRID - 1)
    def _phase2():
        s0 = 1.0 + eps_ref[0]
        s1 = 1.0 + eps_ref[1]
        w0 = w0_ref[...]
        b0 = b0_ref[...]
        w1 = w1_ref[...]
        b1 = b1_ref[...]
        x_all = x_scr[...]
        es = []
        for i in range(M):
            ai = a_scr[i]
            xi = x_all[:, 16 * i:16 * (i + 1)]
            agg0 = jnp.dot(ai, xi, preferred_element_type=jnp.float32)
            hp = jnp.maximum(jnp.dot(s0 * xi + agg0, w0,
                                     preferred_element_type=jnp.float32) + b0, 0.0)
            hm = jnp.maximum(jnp.dot(agg0 - s0 * xi, w0,
                                     preferred_element_type=jnp.float32) + b0, 0.0)
            h2 = jnp.concatenate([hp, hm], axis=1)
            agg1 = jnp.dot(ai, h2, preferred_element_type=jnp.float32)
            ep = jnp.dot(s1 * hp + agg1[:, :HID], w1,
                         preferred_element_type=jnp.float32) + b1
            em = jnp.dot(s1 * hm - agg1[:, HID:], w1,
                         preferred_element_type=jnp.float32) + b1
            es.append(ep + em)
        xcat = jnp.concatenate(es, axis=1)
        hmid = jnp.maximum(jnp.dot(xcat, rw0_ref[...],
                                   preferred_element_type=jnp.float32)
                           + rb0_ref[...], 0.0)
        out_ref[...] = jnp.dot(hmid, rw1_ref[...],
                               preferred_element_type=jnp.float32) + rb1_ref[...]


def _full(shape):
    nd = len(shape)
    return pl.BlockSpec(shape, lambda i: (0,) * nd)


@jax.jit
def _run(g2, enc_W0, enc_b0, enc_W1, enc_b1, enc_eps,
         rho_W0, rho_b0, rho_W1, rho_b1):
    return pl.pallas_call(
        _body,
        grid=(GRID,),
        in_specs=[
            pl.BlockSpec(memory_space=pltpu.SMEM),
            pl.BlockSpec((BU, N, M * D), lambda i: (i, 0, 0)),
            _full((D, HID)), _full((1, HID)),
            _full((HID, OUT)), _full((1, OUT)),
            _full((M * OUT, HID)), _full((1, HID)),
            _full((HID, OUT)), _full((1, OUT)),
        ],
        out_specs=pl.BlockSpec((N, OUT), lambda i: (0, 0)),
        out_shape=jax.ShapeDtypeStruct((N, OUT), jnp.float32),
        scratch_shapes=[
            pltpu.VMEM((M, N, N), jnp.float32),
            pltpu.VMEM((N, M * D), jnp.float32),
        ],
        compiler_params=pltpu.CompilerParams(
            dimension_semantics=("arbitrary",),
        ),
    )(enc_eps, g2, enc_W0, enc_b0, enc_W1, enc_b1,
      rho_W0, rho_b0, rho_W1, rho_b1)


def kernel(g, enc_W0, enc_b0, enc_W1, enc_b1, enc_eps,
           rho_W0, rho_b0, rho_W1, rho_b1):
    g2 = g.reshape(N, N, M * D)
    out = _run(g2, enc_W0, enc_b0.reshape(1, HID),
               enc_W1, enc_b1.reshape(1, OUT), enc_eps,
               rho_W0, rho_b0.reshape(1, HID),
               rho_W1, rho_b1.reshape(1, OUT))
    return out[None]


# speedup vs baseline: 4.5839x; 2.3303x over previous
import functools

import jax
import jax.numpy as jnp
from jax.experimental import pallas as pl
from jax.experimental.pallas import tpu as pltpu

N = 512
M = 4
D = 16
HID = 32
OUT = 16
BU = 32
GRID = N // BU


def _body(eps_ref, g_ref, w0_ref, b0_ref, w1_ref, b1_ref,
          rw0_ref, rb0_ref, rw1_ref, rb1_ref, out_ref,
          a_scr, x_scr):
    step = pl.program_id(0)
    u0 = step * BU

    blk = g_ref[...]
    x_scr[pl.ds(u0, BU), :] = blk[:, 0, :]
    blk_t = jnp.swapaxes(blk, 1, 2)
    for c in range(M):
        a_scr[c, pl.ds(u0, BU), :] = jnp.sum(
            blk_t[:, 16 * c:16 * (c + 1), :], axis=1) * (1.0 / D)

    @pl.when(step == GRID - 1)
    def _phase2():
        s0 = 1.0 + eps_ref[0]
        s1 = 1.0 + eps_ref[1]
        w0 = w0_ref[...]
        b0 = b0_ref[...]
        w1 = w1_ref[...]
        b1 = b1_ref[...]
        x_all = x_scr[...]
        es = []
        for i in range(M):
            ai = a_scr[i]
            xi = x_all[:, 16 * i:16 * (i + 1)]
            agg0 = jnp.dot(ai, xi, preferred_element_type=jnp.float32)
            hp = jnp.maximum(jnp.dot(s0 * xi + agg0, w0,
                                     preferred_element_type=jnp.float32) + b0, 0.0)
            hm = jnp.maximum(jnp.dot(agg0 - s0 * xi, w0,
                                     preferred_element_type=jnp.float32) + b0, 0.0)
            h2 = jnp.concatenate([hp, hm], axis=1)
            agg1 = jnp.dot(ai, h2, preferred_element_type=jnp.float32)
            ep = jnp.dot(s1 * hp + agg1[:, :HID], w1,
                         preferred_element_type=jnp.float32) + b1
            em = jnp.dot(s1 * hm - agg1[:, HID:], w1,
                         preferred_element_type=jnp.float32) + b1
            es.append(ep + em)
        xcat = jnp.concatenate(es, axis=1)
        hmid = jnp.maximum(jnp.dot(xcat, rw0_ref[...],
                                   preferred_element_type=jnp.float32)
                           + rb0_ref[...], 0.0)
        out_ref[...] = jnp.dot(hmid, rw1_ref[...],
                               preferred_element_type=jnp.float32) + rb1_ref[...]


def _full(shape):
    nd = len(shape)
    return pl.BlockSpec(shape, lambda i: (0,) * nd)


@jax.jit
def _run(g2, enc_W0, enc_b0, enc_W1, enc_b1, enc_eps,
         rho_W0, rho_b0, rho_W1, rho_b1):
    return pl.pallas_call(
        _body,
        grid=(GRID,),
        in_specs=[
            pl.BlockSpec(memory_space=pltpu.SMEM),
            pl.BlockSpec((BU, N, M * D), lambda i: (i, 0, 0)),
            _full((D, HID)), _full((1, HID)),
            _full((HID, OUT)), _full((1, OUT)),
            _full((M * OUT, HID)), _full((1, HID)),
            _full((HID, OUT)), _full((1, OUT)),
        ],
        out_specs=pl.BlockSpec((N, OUT), lambda i: (0, 0)),
        out_shape=jax.ShapeDtypeStruct((N, OUT), jnp.float32),
        scratch_shapes=[
            pltpu.VMEM((M, N, N), jnp.float32),
            pltpu.VMEM((N, M * D), jnp.float32),
        ],
        compiler_params=pltpu.CompilerParams(
            dimension_semantics=("arbitrary",),
        ),
    )(enc_eps, g2, enc_W0, enc_b0, enc_W1, enc_b1,
      rho_W0, rho_b0, rho_W1, rho_b1)


def kernel(g, enc_W0, enc_b0, enc_W1, enc_b1, enc_eps,
           rho_W0, rho_b0, rho_W1, rho_b1):
    g2 = g.reshape(N, N, M * D)
    out = _run(g2, enc_W0, enc_b0.reshape(1, HID),
               enc_W1, enc_b1.reshape(1, OUT), enc_eps,
               rho_W0, rho_b0.reshape(1, HID),
               rho_W1, rho_b1.reshape(1, OUT))
    return out[None]
